# baseline (speedup 1.0000x reference)
import dataclasses

import jax
import jax.numpy as jnp
from jax import lax
from jax.experimental import pallas as pl
from jax.experimental.pallas import tpu as pltpu
from jax.experimental.pallas import tpu_sc as plsc

_N = 2048
_D = 768
_R = 256
_K = 5
_SETS = 4
_PAD = 16

_INF = float("inf")


def _d2_body(x_blk_ref, xt_ref, out_ref):
    x_blk = x_blk_ref[...]
    xt = xt_ref[...]
    g = lax.dot_general(
        x_blk, xt, (((1,), (0,)), ((), ())),
        preferred_element_type=jnp.float32,
        precision=lax.Precision.DEFAULT,
    )
    sq_r = jnp.sum(x_blk * x_blk, axis=1, keepdims=True)
    sq_c = jnp.sum(xt * xt, axis=0, keepdims=True)
    out_ref[...] = sq_r + sq_c - 2.0 * g


def _d2_producer(x, xt):
    return pl.pallas_call(
        _d2_body,
        grid=(_N // _R,),
        in_specs=[
            pl.BlockSpec((_R, _D), lambda i: (i, 0)),
            pl.BlockSpec((_D, _N), lambda i: (0, 0)),
        ],
        out_specs=pl.BlockSpec((_R, _N), lambda i: (i, 0)),
        out_shape=jax.ShapeDtypeStruct((_N, _N), jnp.float32),
    )(x, xt)


def _sc_row_body(in_vmem, out_vmem):
    nslices = _N // 16
    acc = [[jnp.full((16,), _INF, jnp.float32) for _ in range(_K)]
           for _ in range(_SETS)]

    def insert(aset, v):
        for j in range(_K):
            lo = jnp.minimum(aset[j], v)
            v = jnp.maximum(aset[j], v)
            aset[j] = lo

    for s in range(nslices):
        v = in_vmem[0, pl.ds(s * 16, 16)]
        insert(acc[s % _SETS], v)

    for p in range(1, _SETS):
        for j in range(_K):
            insert(acc[0], acc[p][j])
    a = acc[0]

    lane = lax.iota(jnp.int32, 16)
    z = jnp.full((16,), _INF, jnp.float32)
    for t in range(_K):
        m = jnp.min(a[0])
        z = jnp.where(lane == t, m, z)
        if t < _K - 1:
            hit = a[0] <= m
            for j in range(_K - 1):
                a[j] = jnp.where(hit, a[j + 1], a[j])
            a[_K - 1] = jnp.where(hit, _INF, a[_K - 1])
    out_vmem[0, :] = z


def _sc_selector(d2):
    mesh = plsc.VectorSubcoreMesh(core_axis_name="core",
                                  subcore_axis_name="subcore")
    cp = pltpu.CompilerParams()
    if "needs_layout_passes" in pltpu.CompilerParams.__dataclass_fields__:
        cp = dataclasses.replace(cp, needs_layout_passes=False)

    @pl.kernel(out_type=jax.ShapeDtypeStruct((_N, _PAD), jnp.float32),
               mesh=mesh, compiler_params=cp)
    def sel_kernel(d2_hbm, out_hbm):
        pltpu.emit_pipeline(
            _sc_row_body,
            grid=(_N,),
            in_specs=[pl.BlockSpec((1, _N), index_map=lambda i: (i, 0))],
            out_specs=[pl.BlockSpec((1, _PAD), index_map=lambda i: (i, 0))],
            core_axis_name=("core", "subcore"),
            dimension_semantics=(pltpu.PARALLEL,),
        )(d2_hbm, out_hbm)

    return sel_kernel(d2)


def _fin_body(sel_ref, out_ref):
    x = sel_ref[...]
    col = lax.broadcasted_iota(jnp.int32, (_N, _PAD), 1)
    dist = jnp.sqrt(jnp.maximum(x, 0.0) + 1e-12)
    dist = jnp.where(col < _K, dist, 0.0)
    s1 = jnp.sum(dist)
    s2 = jnp.sum(dist * dist)
    cnt = jnp.float32(_N * _K)
    var = (s2 - s1 * s1 / cnt) / (cnt - 1.0)
    out_ref[0, 0] = -jnp.sqrt(jnp.maximum(var, 0.0))


def _finisher(sel):
    return pl.pallas_call(
        _fin_body,
        out_specs=pl.BlockSpec(memory_space=pltpu.SMEM),
        out_shape=jax.ShapeDtypeStruct((1, 1), jnp.float32),
    )(sel)


def kernel(latent):
    x = latent[0]
    xt = x.T
    d2 = _d2_producer(x, xt)
    sel = _sc_selector(d2)
    out = _finisher(sel)
    return out[0, 0]


# speedup vs baseline: 2.0082x; 2.0082x over previous
import jax
import jax.numpy as jnp
from jax.experimental import pallas as pl
from jax.experimental.pallas import tpu as pltpu

_N = 2048
_D = 768
_R = 256
_K = 5


def _body(x_blk_ref, xt_ref, out_ref, sqc_ref, acc_ref):
    i = pl.program_id(0)
    nblk = pl.num_programs(0)

    x_blk = x_blk_ref[...]
    xt = xt_ref[...]

    @pl.when(i == 0)
    def _():
        sqc_ref[...] = jnp.sum(xt * xt, axis=0, keepdims=True)
        acc_ref[0] = 0.0
        acc_ref[1] = 0.0

    g = jax.lax.dot_general(
        x_blk, xt, (((1,), (0,)), ((), ())),
        preferred_element_type=jnp.float32,
        precision=jax.lax.Precision.DEFAULT,
    )
    sq_r = jnp.sum(x_blk * x_blk, axis=1, keepdims=True)
    d2 = sq_r + sqc_ref[...] - 2.0 * g

    ms = []
    for t in range(_K):
        m = jnp.min(d2, axis=1, keepdims=True)
        ms.append(m)
        if t < _K - 1:
            d2 = jnp.where(d2 <= m, jnp.float32(jnp.inf), d2)

    msel = jnp.concatenate(ms, axis=1)
    d2sel = jnp.maximum(msel, 0.0) + 1e-12
    s = jnp.sum(jnp.sqrt(d2sel))
    ss = jnp.sum(d2sel)

    acc_ref[0] += s
    acc_ref[1] += ss

    @pl.when(i == nblk - 1)
    def _():
        cnt = jnp.float32(_N * _K)
        s1 = acc_ref[0]
        s2 = acc_ref[1]
        var = (s2 - s1 * s1 / cnt) / (cnt - 1.0)
        out_ref[0, 0] = -jnp.sqrt(jnp.maximum(var, 0.0))


def kernel(latent):
    x = latent[0]
    xt = x.T
    out = pl.pallas_call(
        _body,
        grid=(_N // _R,),
        in_specs=[
            pl.BlockSpec((_R, _D), lambda i: (i, 0)),
            pl.BlockSpec((_D, _N), lambda i: (0, 0)),
        ],
        out_specs=pl.BlockSpec((1, 1), lambda i: (0, 0),
                               memory_space=pltpu.SMEM),
        out_shape=jax.ShapeDtypeStruct((1, 1), jnp.float32),
        scratch_shapes=[pltpu.VMEM((1, _N), jnp.float32),
                        pltpu.SMEM((2,), jnp.float32)],
    )(x, xt)
    return out[0, 0]
